# baseline (speedup 1.0000x reference)
import jax
import jax.numpy as jnp
from jax import lax
from jax.experimental import pallas as pl
from jax.experimental.pallas import tpu as pltpu
from jax.experimental.pallas import tpu_sc as plsc

STATE = 16384
NCONN = 8192
PDIM = 8192
KBLK = 512

NWORK = 32
CHUNK = STATE // NWORK
LANES = 16


def _mv_body(m_ref, w_ref, o_ref):
    @pl.when(pl.program_id(0) == 0)
    def _():
        o_ref[...] = jnp.zeros_like(o_ref)

    o_ref[...] += jnp.sum(m_ref[...] * w_ref[...], axis=0, keepdims=True)


def _matvec(mesa, w):
    out = pl.pallas_call(
        _mv_body,
        grid=(NCONN // KBLK,),
        in_specs=[
            pl.BlockSpec((KBLK, 1), lambda k: (k, 0)),
            pl.BlockSpec((KBLK, PDIM), lambda k: (k, 0)),
        ],
        out_specs=pl.BlockSpec((1, PDIM), lambda k: (0, 0)),
        out_shape=jax.ShapeDtypeStruct((1, PDIM), jnp.float32),
    )(mesa.reshape(NCONN, 1), w)
    return out.reshape(PDIM)


def _sc_body(vals_hbm, idx_hbm, bias_hbm, out_hbm, idx_v, vals_v, buf_v):
    cid = lax.axis_index("c")
    sid = lax.axis_index("s")
    wid = sid * 2 + cid
    base = wid * CHUNK
    pltpu.sync_copy(bias_hbm.at[pl.ds(base, CHUNK)], buf_v)
    pltpu.sync_copy(idx_hbm, idx_v)
    pltpu.sync_copy(vals_hbm, vals_v)

    UNROLL = 8

    def body(i, carry):
        for u in range(UNROLL):
            off = (i * UNROLL + u) * LANES
            vi = idx_v[pl.ds(off, LANES)]
            vv = vals_v[pl.ds(off, LANES)]
            rel = vi - base
            m = (rel >= 0) & (rel < CHUNK)
            plsc.addupdate_scatter(buf_v, [rel], vv, mask=m)
        return carry

    lax.fori_loop(0, NCONN // (LANES * UNROLL), body, 0)
    pltpu.sync_copy(buf_v, out_hbm.at[pl.ds(base, CHUNK)])


def _sc_scatter(vals, conn_idx, bias):
    run = pl.kernel(
        _sc_body,
        out_type=jax.ShapeDtypeStruct((STATE,), jnp.float32),
        mesh=plsc.VectorSubcoreMesh(core_axis_name="c", subcore_axis_name="s"),
        scratch_types=[
            pltpu.VMEM((NCONN,), jnp.int32),
            pltpu.VMEM((NCONN,), jnp.float32),
            pltpu.VMEM((CHUNK,), jnp.float32),
        ],
        compiler_params=pltpu.CompilerParams(needs_layout_passes=False),
    )
    return run(vals, conn_idx, bias)


def kernel(mesa_parameter, meta_weight, meta_bias, conn_idx):
    vals = _matvec(mesa_parameter, meta_weight)
    return _sc_scatter(vals, conn_idx, meta_bias)


# speedup vs baseline: 1.0261x; 1.0261x over previous
import jax
import jax.numpy as jnp
from jax import lax
from jax.experimental import pallas as pl
from jax.experimental.pallas import tpu as pltpu
from jax.experimental.pallas import tpu_sc as plsc

STATE = 16384
NCONN = 8192
PDIM = 8192
KBLK = 256

NWORK = 32
CHUNK = STATE // NWORK
LANES = 16


def _mv_body(m_ref, w_ref, o_ref):
    @pl.when(pl.program_id(0) == 0)
    def _():
        o_ref[...] = jnp.zeros_like(o_ref)

    o_ref[...] += jnp.sum(m_ref[...] * w_ref[...], axis=0, keepdims=True)


def _matvec(mesa, w):
    out = pl.pallas_call(
        _mv_body,
        grid=(NCONN // KBLK,),
        in_specs=[
            pl.BlockSpec((KBLK, 1), lambda k: (k, 0)),
            pl.BlockSpec((KBLK, PDIM), lambda k: (k, 0)),
        ],
        out_specs=pl.BlockSpec((1, PDIM), lambda k: (0, 0)),
        out_shape=jax.ShapeDtypeStruct((1, PDIM), jnp.float32),
    )(mesa.reshape(NCONN, 1), w)
    return out.reshape(PDIM)


def _sc_body(vals_hbm, idx_hbm, bias_hbm, out_hbm, idx_v, vals_v, buf_v):
    cid = lax.axis_index("c")
    sid = lax.axis_index("s")
    wid = sid * 2 + cid
    base = wid * CHUNK
    pltpu.sync_copy(bias_hbm.at[pl.ds(base, CHUNK)], buf_v)
    pltpu.sync_copy(idx_hbm, idx_v)
    pltpu.sync_copy(vals_hbm, vals_v)

    UNROLL = 8

    def body(i, carry):
        for u in range(UNROLL):
            off = (i * UNROLL + u) * LANES
            vi = idx_v[pl.ds(off, LANES)]
            vv = vals_v[pl.ds(off, LANES)]
            rel = vi - base
            m = (rel >= 0) & (rel < CHUNK)
            plsc.addupdate_scatter(buf_v, [rel], vv, mask=m)
        return carry

    lax.fori_loop(0, NCONN // (LANES * UNROLL), body, 0)
    pltpu.sync_copy(buf_v, out_hbm.at[pl.ds(base, CHUNK)])


def _sc_scatter(vals, conn_idx, bias):
    run = pl.kernel(
        _sc_body,
        out_type=jax.ShapeDtypeStruct((STATE,), jnp.float32),
        mesh=plsc.VectorSubcoreMesh(core_axis_name="c", subcore_axis_name="s"),
        scratch_types=[
            pltpu.VMEM((NCONN,), jnp.int32),
            pltpu.VMEM((NCONN,), jnp.float32),
            pltpu.VMEM((CHUNK,), jnp.float32),
        ],
        compiler_params=pltpu.CompilerParams(needs_layout_passes=False),
    )
    return run(vals, conn_idx, bias)


def kernel(mesa_parameter, meta_weight, meta_bias, conn_idx):
    vals = _matvec(mesa_parameter, meta_weight)
    return _sc_scatter(vals, conn_idx, meta_bias)
